# baseline (speedup 1.0000x reference)
import jax
import jax.numpy as jnp
from jax import lax
from jax.experimental import pallas as pl
from jax.experimental.pallas import tpu as pltpu
from jax.experimental.pallas import tpu_sc as plsc

N = 10000
D = 128
E = 320000

NC = 2
NS = 16
NW = NC * NS
CH = 80
CH0 = 40
CH1 = 2 * CH - CH0
EPW = CH * 128
E_PAD = NW * EPW
N_PAD = 10240
RPT = N_PAD // NS
G = 8
BN = 2000


def _zero_rows_f32(ref, ncols):
    def zr(r, _):
        def zc(k, _):
            ref[r, pl.ds(k * 16, 16)] = jnp.zeros((16,), jnp.float32)
            return 0
        return lax.fori_loop(0, ncols // 16, zc, 0)
    lax.fori_loop(0, ref.shape[0], zr, 0)


def _zero_shared_slice(zbuf, shared, base):
    nfull = RPT // 128
    for t in range(nfull):
        pltpu.sync_copy(zbuf, shared.at[pl.ds(base + t * 128, 128)])
    rem = RPT - nfull * 128
    if rem:
        pltpu.sync_copy(zbuf.at[pl.ds(0, rem)],
                        shared.at[pl.ds(base + nfull * 128, rem)])


def _make_edge_kernel():
    mesh = plsc.VectorSubcoreMesh(core_axis_name="c", subcore_axis_name="s",
                                  num_cores=NC, num_subcores=NS)
    out_type = jax.ShapeDtypeStruct((NC, N_PAD, D), jnp.float32)
    scratch = [
        pltpu.VMEM((G, 128), jnp.int32),
        pltpu.VMEM((G, 128), jnp.int32),
        pltpu.VMEM((128, D), jnp.float32),
        pltpu.VMEM((128, D), jnp.float32),
        pltpu.SemaphoreType.DMA,
        pltpu.SemaphoreType.DMA,
        pltpu.VMEM_SHARED((N_PAD, D), jnp.float32),
    ]

    def body(xfeat, srcm, dstm, out, sidx, didx, rows0, rows1, sem0, sem1,
             acc):
        c = lax.axis_index("c")
        s = lax.axis_index("s")
        w = c * NS + s
        base = s * RPT

        _zero_rows_f32(rows0, D)
        _zero_shared_slice(rows0, acc, base)
        plsc.subcore_barrier()

        bufs = (rows0, rows1)
        sems = (sem0, sem1)
        nch = lax.select(c == 0, CH0 // G, CH1 // G)
        cbase = lax.select(c == 0, s * CH0, NS * CH0 + s * CH1)

        def gbody(g, _):
            pltpu.sync_copy(srcm.at[pl.ds(cbase + g * G, G)], sidx)
            pltpu.sync_copy(dstm.at[pl.ds(cbase + g * G, G)], didx)
            pltpu.async_copy(xfeat.at[sidx.at[0]], bufs[0], sems[0])
            for k in range(G):
                b = k % 2
                pltpu.make_async_copy(xfeat.at[sidx.at[k]], bufs[b],
                                      sems[b]).wait()
                if k + 1 < G:
                    nb = (k + 1) % 2
                    pltpu.async_copy(xfeat.at[sidx.at[k + 1]], bufs[nb],
                                     sems[nb])
                pltpu.sync_copy(bufs[b], acc.at[didx.at[k]], add=True)
            return 0
        lax.fori_loop(0, nch, gbody, 0)
        plsc.subcore_barrier()

        sl = pl.ds(base, RPT)
        pltpu.sync_copy(acc.at[sl], out.at[c, sl])

    cp = pltpu.CompilerParams(needs_layout_passes=False)
    return pl.kernel(body, out_type=out_type, mesh=mesh,
                     scratch_types=scratch, compiler_params=cp)


def _make_count_kernel():
    mesh = plsc.VectorSubcoreMesh(core_axis_name="c", subcore_axis_name="s",
                                  num_cores=NC, num_subcores=NS)
    out_type = jax.ShapeDtypeStruct((NC, 1, N_PAD), jnp.float32)
    scratch = [
        pltpu.VMEM((G, 128), jnp.int32),
        pltpu.VMEM((N_PAD,), jnp.float32),
        pltpu.VMEM((NS, 128), jnp.float32),
        pltpu.VMEM_SHARED((NS, N_PAD), jnp.float32),
    ]

    def body(dstm, cnt_out, didx, cntv, tbuf, sh):
        c = lax.axis_index("c")
        s = lax.axis_index("s")
        w = c * NS + s

        def zc1(k, _):
            cntv[pl.ds(k * 16, 16)] = jnp.zeros((16,), jnp.float32)
            return 0
        lax.fori_loop(0, N_PAD // 16, zc1, 0)

        one16 = jnp.ones((16,), jnp.float32)

        def gbody(g, _):
            pltpu.sync_copy(dstm.at[pl.ds(w * CH + g * G, G)], didx)

            def ebody(j, _):
                for k in range(128 // 16):
                    idx16 = didx[j, pl.ds(k * 16, 16)]
                    plsc.addupdate_scatter(cntv, [idx16], one16)
                return 0
            lax.fori_loop(0, G, ebody, 0)
            return 0
        lax.fori_loop(0, CH // G, gbody, 0)

        pltpu.sync_copy(cntv, sh.at[s])
        plsc.subcore_barrier()
        for h in range(5):
            pltpu.sync_copy(sh.at[:, pl.ds(s * 640 + h * 128, 128)], tbuf)
            for g in range(8):
                csl = pl.ds(g * 16, 16)
                v = tbuf[0, csl]
                for r in range(1, NS):
                    v = v + tbuf[r, csl]
                cntv[pl.ds(h * 128 + g * 16, 16)] = v
        pltpu.sync_copy(cntv.at[pl.ds(0, 640)],
                        cnt_out.at[c, 0, pl.ds(s * 640, 640)])

    cp = pltpu.CompilerParams(needs_layout_passes=False)
    return pl.kernel(body, out_type=out_type, mesh=mesh,
                     scratch_types=scratch, compiler_params=cp)


_sc_kernel_cache = {}


def _sc_kernel(kind):
    if kind not in _sc_kernel_cache:
        maker = {"edge": _make_edge_kernel, "count": _make_count_kernel}
        _sc_kernel_cache[kind] = maker[kind]()
    return _sc_kernel_cache[kind]


def _mm2_body(x_ref, wl_ref, wr_ref, b_ref, yl_ref, yr_ref):
    xb = x_ref[...]
    yl_ref[...] = jnp.dot(xb, wl_ref[...], preferred_element_type=jnp.float32)
    yr_ref[...] = (jnp.dot(xb, wr_ref[...], preferred_element_type=jnp.float32)
                   + b_ref[...])


_mm2 = pl.pallas_call(
    _mm2_body,
    grid=(N // BN,),
    in_specs=[
        pl.BlockSpec((BN, D), lambda i: (i, 0)),
        pl.BlockSpec((D, D), lambda i: (0, 0)),
        pl.BlockSpec((D, D), lambda i: (0, 0)),
        pl.BlockSpec((1, D), lambda i: (0, 0)),
    ],
    out_specs=[pl.BlockSpec((BN, D), lambda i: (i, 0))] * 2,
    out_shape=[jax.ShapeDtypeStruct((N, D), jnp.float32)] * 2,
)


def _part0(i):
    return (0, i, 0)


def _part1(i):
    return (1, i, 0)


def _fuse_body(s0, s1, cn0, cn1, xr, wl, wr, b, hl_ref, hr_ref):
    cnt = cn0[0] + cn1[0]
    recip = 1.0 / jnp.maximum(cnt, 1.0)
    h = jnp.maximum((s0[0] + s1[0]) * recip + xr[...], 0.0)
    hl_ref[...] = jnp.dot(h, wl[...], preferred_element_type=jnp.float32)
    hr_ref[...] = (jnp.dot(h, wr[...], preferred_element_type=jnp.float32)
                   + b[...])


_fuse = pl.pallas_call(
    _fuse_body,
    grid=(N // BN,),
    in_specs=[
        pl.BlockSpec((1, BN, D), _part0),
        pl.BlockSpec((1, BN, D), _part1),
        pl.BlockSpec((1, BN, 1), _part0),
        pl.BlockSpec((1, BN, 1), _part1),
        pl.BlockSpec((BN, D), lambda i: (i, 0)),
        pl.BlockSpec((D, D), lambda i: (0, 0)),
        pl.BlockSpec((D, D), lambda i: (0, 0)),
        pl.BlockSpec((1, D), lambda i: (0, 0)),
    ],
    out_specs=[pl.BlockSpec((BN, D), lambda i: (i, 0))] * 2,
    out_shape=[jax.ShapeDtypeStruct((N, D), jnp.float32)] * 2,
)


def _out_body(s0, s1, cn0, cn1, hr, o_ref):
    cnt = cn0[0] + cn1[0]
    recip = 1.0 / jnp.maximum(cnt, 1.0)
    v = (s0[0] + s1[0]) * recip + hr[...]
    z = v - jnp.max(v, axis=1, keepdims=True)
    o_ref[...] = z - jnp.log(jnp.sum(jnp.exp(z), axis=1, keepdims=True))


_outk = pl.pallas_call(
    _out_body,
    grid=(N // BN,),
    in_specs=[
        pl.BlockSpec((1, BN, D), _part0),
        pl.BlockSpec((1, BN, D), _part1),
        pl.BlockSpec((1, BN, 1), _part0),
        pl.BlockSpec((1, BN, 1), _part1),
        pl.BlockSpec((BN, D), lambda i: (i, 0)),
    ],
    out_specs=pl.BlockSpec((BN, D), lambda i: (i, 0)),
    out_shape=jax.ShapeDtypeStruct((N, D), jnp.float32),
)


def kernel(x, edge_index, W1l, b1, W1r, W2l, b2, W2r):
    src = edge_index[0]
    dst = edge_index[1]
    pad = E_PAD - E
    srcm = jnp.concatenate([src, jnp.zeros((pad,), jnp.int32)])
    srcm = srcm.reshape(E_PAD // 128, 128)
    dstm = jnp.concatenate([dst, jnp.full((pad,), N, jnp.int32)])
    dstm = dstm.reshape(E_PAD // 128, 128)

    xl, xr = _mm2(x, W1l.T, W1r.T, b1.reshape(1, D))
    c_all = _sc_kernel("count")(dstm)
    s_all = _sc_kernel("edge")(xl, srcm, dstm)
    cn3 = c_all.reshape(NC, N_PAD, 1)
    hl, hr = _fuse(s_all, s_all, cn3, cn3, xr, W2l.T, W2r.T,
                   b2.reshape(1, D))
    t_all = _sc_kernel("edge")(hl, srcm, dstm)
    return _outk(t_all, t_all, cn3, cn3, hr)


# speedup vs baseline: 1.1590x; 1.1590x over previous
import jax
import jax.numpy as jnp
from jax import lax
from jax.experimental import pallas as pl
from jax.experimental.pallas import tpu as pltpu
from jax.experimental.pallas import tpu_sc as plsc

N = 10000
D = 128
E = 320000

NC = 2
NS = 16
NW = NC * NS
CH = 80
CH0 = 120
CH1 = 2 * CH - CH0
EPW = CH * 128
E_PAD = NW * EPW
N_PAD = 10240
RPT = N_PAD // NS
G = 8
BN = 2000


def _zero_rows_f32(ref, ncols):
    def zr(r, _):
        def zc(k, _):
            ref[r, pl.ds(k * 16, 16)] = jnp.zeros((16,), jnp.float32)
            return 0
        return lax.fori_loop(0, ncols // 16, zc, 0)
    lax.fori_loop(0, ref.shape[0], zr, 0)


def _zero_shared_slice(zbuf, shared, base):
    nfull = RPT // 128
    for t in range(nfull):
        pltpu.sync_copy(zbuf, shared.at[pl.ds(base + t * 128, 128)])
    rem = RPT - nfull * 128
    if rem:
        pltpu.sync_copy(zbuf.at[pl.ds(0, rem)],
                        shared.at[pl.ds(base + nfull * 128, rem)])


def _make_edge_kernel():
    mesh = plsc.VectorSubcoreMesh(core_axis_name="c", subcore_axis_name="s",
                                  num_cores=NC, num_subcores=NS)
    out_type = jax.ShapeDtypeStruct((NC, N_PAD, D), jnp.float32)
    scratch = [
        pltpu.VMEM((G, 128), jnp.int32),
        pltpu.VMEM((G, 128), jnp.int32),
        pltpu.VMEM((128, D), jnp.float32),
        pltpu.VMEM((128, D), jnp.float32),
        pltpu.SemaphoreType.DMA,
        pltpu.SemaphoreType.DMA,
        pltpu.VMEM_SHARED((N_PAD, D), jnp.float32),
    ]

    def body(xfeat, srcm, dstm, out, sidx, didx, rows0, rows1, sem0, sem1,
             acc):
        c = lax.axis_index("c")
        s = lax.axis_index("s")
        w = c * NS + s
        base = s * RPT

        _zero_rows_f32(rows0, D)
        _zero_shared_slice(rows0, acc, base)
        plsc.subcore_barrier()

        bufs = (rows0, rows1)
        sems = (sem0, sem1)
        nch = lax.select(c == 0, CH0 // G, CH1 // G)
        cbase = lax.select(c == 0, s * CH0, NS * CH0 + s * CH1)

        def gbody(g, _):
            pltpu.sync_copy(srcm.at[pl.ds(cbase + g * G, G)], sidx)
            pltpu.sync_copy(dstm.at[pl.ds(cbase + g * G, G)], didx)
            pltpu.async_copy(xfeat.at[sidx.at[0]], bufs[0], sems[0])
            for k in range(G):
                b = k % 2
                pltpu.make_async_copy(xfeat.at[sidx.at[k]], bufs[b],
                                      sems[b]).wait()
                if k + 1 < G:
                    nb = (k + 1) % 2
                    pltpu.async_copy(xfeat.at[sidx.at[k + 1]], bufs[nb],
                                     sems[nb])
                pltpu.sync_copy(bufs[b], acc.at[didx.at[k]], add=True)
            return 0
        lax.fori_loop(0, nch, gbody, 0)
        plsc.subcore_barrier()

        sl = pl.ds(base, RPT)
        pltpu.sync_copy(acc.at[sl], out.at[c, sl])

    cp = pltpu.CompilerParams(needs_layout_passes=False)
    return pl.kernel(body, out_type=out_type, mesh=mesh,
                     scratch_types=scratch, compiler_params=cp)


def _make_count_kernel():
    mesh = plsc.VectorSubcoreMesh(core_axis_name="c", subcore_axis_name="s",
                                  num_cores=NC, num_subcores=NS)
    out_type = jax.ShapeDtypeStruct((NC, 1, N_PAD), jnp.float32)
    scratch = [
        pltpu.VMEM((G, 128), jnp.int32),
        pltpu.VMEM((N_PAD,), jnp.float32),
        pltpu.VMEM((NS, 128), jnp.float32),
        pltpu.VMEM_SHARED((NS, N_PAD), jnp.float32),
    ]

    def body(dstm, cnt_out, didx, cntv, tbuf, sh):
        c = lax.axis_index("c")
        s = lax.axis_index("s")
        w = c * NS + s

        def zc1(k, _):
            cntv[pl.ds(k * 16, 16)] = jnp.zeros((16,), jnp.float32)
            return 0
        lax.fori_loop(0, N_PAD // 16, zc1, 0)

        one16 = jnp.ones((16,), jnp.float32)

        def gbody(g, _):
            pltpu.sync_copy(dstm.at[pl.ds(w * CH + g * G, G)], didx)

            def ebody(j, _):
                for k in range(128 // 16):
                    idx16 = didx[j, pl.ds(k * 16, 16)]
                    plsc.addupdate_scatter(cntv, [idx16], one16)
                return 0
            lax.fori_loop(0, G, ebody, 0)
            return 0
        lax.fori_loop(0, CH // G, gbody, 0)

        pltpu.sync_copy(cntv, sh.at[s])
        plsc.subcore_barrier()
        for h in range(5):
            pltpu.sync_copy(sh.at[:, pl.ds(s * 640 + h * 128, 128)], tbuf)
            for g in range(8):
                csl = pl.ds(g * 16, 16)
                v = tbuf[0, csl]
                for r in range(1, NS):
                    v = v + tbuf[r, csl]
                cntv[pl.ds(h * 128 + g * 16, 16)] = v
        pltpu.sync_copy(cntv.at[pl.ds(0, 640)],
                        cnt_out.at[c, 0, pl.ds(s * 640, 640)])

    cp = pltpu.CompilerParams(needs_layout_passes=False)
    return pl.kernel(body, out_type=out_type, mesh=mesh,
                     scratch_types=scratch, compiler_params=cp)


_sc_kernel_cache = {}


def _sc_kernel(kind):
    if kind not in _sc_kernel_cache:
        maker = {"edge": _make_edge_kernel, "count": _make_count_kernel}
        _sc_kernel_cache[kind] = maker[kind]()
    return _sc_kernel_cache[kind]


def _mm2_body(x_ref, wl_ref, wr_ref, b_ref, yl_ref, yr_ref):
    xb = x_ref[...]
    yl_ref[...] = jnp.dot(xb, wl_ref[...], preferred_element_type=jnp.float32)
    yr_ref[...] = (jnp.dot(xb, wr_ref[...], preferred_element_type=jnp.float32)
                   + b_ref[...])


_mm2 = pl.pallas_call(
    _mm2_body,
    grid=(N // BN,),
    in_specs=[
        pl.BlockSpec((BN, D), lambda i: (i, 0)),
        pl.BlockSpec((D, D), lambda i: (0, 0)),
        pl.BlockSpec((D, D), lambda i: (0, 0)),
        pl.BlockSpec((1, D), lambda i: (0, 0)),
    ],
    out_specs=[pl.BlockSpec((BN, D), lambda i: (i, 0))] * 2,
    out_shape=[jax.ShapeDtypeStruct((N, D), jnp.float32)] * 2,
)


def _part0(i):
    return (0, i, 0)


def _part1(i):
    return (1, i, 0)


def _fuse_body(s0, s1, cn0, cn1, xr, wl, wr, b, hl_ref, hr_ref):
    cnt = cn0[0] + cn1[0]
    recip = 1.0 / jnp.maximum(cnt, 1.0)
    h = jnp.maximum((s0[0] + s1[0]) * recip + xr[...], 0.0)
    hl_ref[...] = jnp.dot(h, wl[...], preferred_element_type=jnp.float32)
    hr_ref[...] = (jnp.dot(h, wr[...], preferred_element_type=jnp.float32)
                   + b[...])


_fuse = pl.pallas_call(
    _fuse_body,
    grid=(N // BN,),
    in_specs=[
        pl.BlockSpec((1, BN, D), _part0),
        pl.BlockSpec((1, BN, D), _part1),
        pl.BlockSpec((1, BN, 1), _part0),
        pl.BlockSpec((1, BN, 1), _part1),
        pl.BlockSpec((BN, D), lambda i: (i, 0)),
        pl.BlockSpec((D, D), lambda i: (0, 0)),
        pl.BlockSpec((D, D), lambda i: (0, 0)),
        pl.BlockSpec((1, D), lambda i: (0, 0)),
    ],
    out_specs=[pl.BlockSpec((BN, D), lambda i: (i, 0))] * 2,
    out_shape=[jax.ShapeDtypeStruct((N, D), jnp.float32)] * 2,
)


def _out_body(s0, s1, cn0, cn1, hr, o_ref):
    cnt = cn0[0] + cn1[0]
    recip = 1.0 / jnp.maximum(cnt, 1.0)
    v = (s0[0] + s1[0]) * recip + hr[...]
    z = v - jnp.max(v, axis=1, keepdims=True)
    o_ref[...] = z - jnp.log(jnp.sum(jnp.exp(z), axis=1, keepdims=True))


_outk = pl.pallas_call(
    _out_body,
    grid=(N // BN,),
    in_specs=[
        pl.BlockSpec((1, BN, D), _part0),
        pl.BlockSpec((1, BN, D), _part1),
        pl.BlockSpec((1, BN, 1), _part0),
        pl.BlockSpec((1, BN, 1), _part1),
        pl.BlockSpec((BN, D), lambda i: (i, 0)),
    ],
    out_specs=pl.BlockSpec((BN, D), lambda i: (i, 0)),
    out_shape=jax.ShapeDtypeStruct((N, D), jnp.float32),
)


def kernel(x, edge_index, W1l, b1, W1r, W2l, b2, W2r):
    src = edge_index[0]
    dst = edge_index[1]
    pad = E_PAD - E
    srcm = jnp.concatenate([src, jnp.zeros((pad,), jnp.int32)])
    srcm = srcm.reshape(E_PAD // 128, 128)
    dstm = jnp.concatenate([dst, jnp.full((pad,), N, jnp.int32)])
    dstm = dstm.reshape(E_PAD // 128, 128)

    xl, xr = _mm2(x, W1l.T, W1r.T, b1.reshape(1, D))
    c_all = _sc_kernel("count")(dstm)
    s_all = _sc_kernel("edge")(xl, srcm, dstm)
    cn3 = c_all.reshape(NC, N_PAD, 1)
    hl, hr = _fuse(s_all, s_all, cn3, cn3, xr, W2l.T, W2r.T,
                   b2.reshape(1, D))
    t_all = _sc_kernel("edge")(hl, srcm, dstm)
    return _outk(t_all, t_all, cn3, cn3, hr)


# speedup vs baseline: 1.1777x; 1.0161x over previous
import jax
import jax.numpy as jnp
from jax import lax
from jax.experimental import pallas as pl
from jax.experimental.pallas import tpu as pltpu
from jax.experimental.pallas import tpu_sc as plsc

N = 10000
D = 128
E = 320000

NC = 2
NS = 16
NW = NC * NS
CH = 80
CR = 64
GG = 16
NBUF = 4
CW0 = 240
CW1 = 80
EPW = CH * 128
E_PAD = NW * EPW
N_PAD = 10240
RPT = N_PAD // NS
G = 8
BN = 2000


def _zero_rows_f32(ref, ncols):
    def zr(r, _):
        def zc(k, _):
            ref[r, pl.ds(k * 16, 16)] = jnp.zeros((16,), jnp.float32)
            return 0
        return lax.fori_loop(0, ncols // 16, zc, 0)
    lax.fori_loop(0, ref.shape[0], zr, 0)


def _zero_shared_slice(zbuf, shared, base):
    nfull = RPT // 128
    for t in range(nfull):
        pltpu.sync_copy(zbuf, shared.at[pl.ds(base + t * 128, 128)])
    rem = RPT - nfull * 128
    if rem:
        pltpu.sync_copy(zbuf.at[pl.ds(0, rem)],
                        shared.at[pl.ds(base + nfull * 128, rem)])


def _make_edge_kernel():
    mesh = plsc.VectorSubcoreMesh(core_axis_name="c", subcore_axis_name="s",
                                  num_cores=NC, num_subcores=NS)
    out_type = jax.ShapeDtypeStruct((NC, N_PAD, D), jnp.float32)
    scratch = [
        pltpu.VMEM((GG, CR), jnp.int32),
        pltpu.VMEM((GG, CR), jnp.int32),
        [pltpu.VMEM((CR, D), jnp.float32)] * NBUF,
        [pltpu.SemaphoreType.DMA] * NBUF,
        pltpu.VMEM_SHARED((N_PAD, D), jnp.float32),
    ]

    def body(xfeat, srcm, dstm, out, sidx, didx, bufs, sems, acc):
        c = lax.axis_index("c")
        s = lax.axis_index("s")
        base = s * RPT

        _zero_rows_f32(bufs[0], D)
        for t in range(RPT // CR):
            pltpu.sync_copy(bufs[0], acc.at[pl.ds(base + t * CR, CR)])
        plsc.subcore_barrier()

        nch = lax.select(c == 0, CW0 // GG, CW1 // GG)
        cbase = lax.select(c == 0, s * CW0, NS * CW0 + s * CW1)

        def gbody(g, _):
            pltpu.sync_copy(srcm.at[pl.ds(cbase + g * GG, GG)], sidx)
            pltpu.sync_copy(dstm.at[pl.ds(cbase + g * GG, GG)], didx)
            for p in range(NBUF - 1):
                pltpu.async_copy(xfeat.at[sidx.at[p]], bufs[p], sems[p])
            for k in range(GG):
                b = k % NBUF
                pltpu.make_async_copy(xfeat.at[sidx.at[k]], bufs[b],
                                      sems[b]).wait()
                if k + NBUF - 1 < GG:
                    nb = (k + NBUF - 1) % NBUF
                    pltpu.async_copy(xfeat.at[sidx.at[k + NBUF - 1]],
                                     bufs[nb], sems[nb])
                pltpu.sync_copy(bufs[b], acc.at[didx.at[k]], add=True)
            return 0
        lax.fori_loop(0, nch, gbody, 0)
        plsc.subcore_barrier()

        sl = pl.ds(base, RPT)
        pltpu.sync_copy(acc.at[sl], out.at[c, sl])

    cp = pltpu.CompilerParams(needs_layout_passes=False)
    return pl.kernel(body, out_type=out_type, mesh=mesh,
                     scratch_types=scratch, compiler_params=cp)


def _make_count_kernel():
    mesh = plsc.VectorSubcoreMesh(core_axis_name="c", subcore_axis_name="s",
                                  num_cores=NC, num_subcores=NS)
    out_type = jax.ShapeDtypeStruct((NC, 1, N_PAD), jnp.float32)
    scratch = [
        pltpu.VMEM((G, 128), jnp.int32),
        pltpu.VMEM((N_PAD,), jnp.float32),
        pltpu.VMEM((NS, 128), jnp.float32),
        pltpu.VMEM_SHARED((NS, N_PAD), jnp.float32),
    ]

    def body(dstm, cnt_out, didx, cntv, tbuf, sh):
        c = lax.axis_index("c")
        s = lax.axis_index("s")
        w = c * NS + s

        def zc1(k, _):
            cntv[pl.ds(k * 16, 16)] = jnp.zeros((16,), jnp.float32)
            return 0
        lax.fori_loop(0, N_PAD // 16, zc1, 0)

        one16 = jnp.ones((16,), jnp.float32)

        def gbody(g, _):
            pltpu.sync_copy(dstm.at[pl.ds(w * CH + g * G, G)], didx)

            def ebody(j, _):
                for k in range(128 // 16):
                    idx16 = didx[j, pl.ds(k * 16, 16)]
                    plsc.addupdate_scatter(cntv, [idx16], one16)
                return 0
            lax.fori_loop(0, G, ebody, 0)
            return 0
        lax.fori_loop(0, CH // G, gbody, 0)

        pltpu.sync_copy(cntv, sh.at[s])
        plsc.subcore_barrier()
        for h in range(5):
            pltpu.sync_copy(sh.at[:, pl.ds(s * 640 + h * 128, 128)], tbuf)
            for g in range(8):
                csl = pl.ds(g * 16, 16)
                v = tbuf[0, csl]
                for r in range(1, NS):
                    v = v + tbuf[r, csl]
                cntv[pl.ds(h * 128 + g * 16, 16)] = v
        pltpu.sync_copy(cntv.at[pl.ds(0, 640)],
                        cnt_out.at[c, 0, pl.ds(s * 640, 640)])

    cp = pltpu.CompilerParams(needs_layout_passes=False)
    return pl.kernel(body, out_type=out_type, mesh=mesh,
                     scratch_types=scratch, compiler_params=cp)


_sc_kernel_cache = {}


def _sc_kernel(kind):
    if kind not in _sc_kernel_cache:
        maker = {"edge": _make_edge_kernel, "count": _make_count_kernel}
        _sc_kernel_cache[kind] = maker[kind]()
    return _sc_kernel_cache[kind]


def _mm2_body(x_ref, wl_ref, wr_ref, b_ref, yl_ref, yr_ref):
    xb = x_ref[...]
    yl_ref[...] = jnp.dot(xb, wl_ref[...], preferred_element_type=jnp.float32)
    yr_ref[...] = (jnp.dot(xb, wr_ref[...], preferred_element_type=jnp.float32)
                   + b_ref[...])


_mm2 = pl.pallas_call(
    _mm2_body,
    grid=(N // BN,),
    in_specs=[
        pl.BlockSpec((BN, D), lambda i: (i, 0)),
        pl.BlockSpec((D, D), lambda i: (0, 0)),
        pl.BlockSpec((D, D), lambda i: (0, 0)),
        pl.BlockSpec((1, D), lambda i: (0, 0)),
    ],
    out_specs=[pl.BlockSpec((BN, D), lambda i: (i, 0))] * 2,
    out_shape=[jax.ShapeDtypeStruct((N, D), jnp.float32)] * 2,
)


def _part0(i):
    return (0, i, 0)


def _part1(i):
    return (1, i, 0)


def _fuse_body(s0, s1, cn0, cn1, xr, wl, wr, b, hl_ref, hr_ref):
    cnt = cn0[0] + cn1[0]
    recip = 1.0 / jnp.maximum(cnt, 1.0)
    h = jnp.maximum((s0[0] + s1[0]) * recip + xr[...], 0.0)
    hl_ref[...] = jnp.dot(h, wl[...], preferred_element_type=jnp.float32)
    hr_ref[...] = (jnp.dot(h, wr[...], preferred_element_type=jnp.float32)
                   + b[...])


_fuse = pl.pallas_call(
    _fuse_body,
    grid=(N // BN,),
    in_specs=[
        pl.BlockSpec((1, BN, D), _part0),
        pl.BlockSpec((1, BN, D), _part1),
        pl.BlockSpec((1, BN, 1), _part0),
        pl.BlockSpec((1, BN, 1), _part1),
        pl.BlockSpec((BN, D), lambda i: (i, 0)),
        pl.BlockSpec((D, D), lambda i: (0, 0)),
        pl.BlockSpec((D, D), lambda i: (0, 0)),
        pl.BlockSpec((1, D), lambda i: (0, 0)),
    ],
    out_specs=[pl.BlockSpec((BN, D), lambda i: (i, 0))] * 2,
    out_shape=[jax.ShapeDtypeStruct((N, D), jnp.float32)] * 2,
)


def _out_body(s0, s1, cn0, cn1, hr, o_ref):
    cnt = cn0[0] + cn1[0]
    recip = 1.0 / jnp.maximum(cnt, 1.0)
    v = (s0[0] + s1[0]) * recip + hr[...]
    z = v - jnp.max(v, axis=1, keepdims=True)
    o_ref[...] = z - jnp.log(jnp.sum(jnp.exp(z), axis=1, keepdims=True))


_outk = pl.pallas_call(
    _out_body,
    grid=(N // BN,),
    in_specs=[
        pl.BlockSpec((1, BN, D), _part0),
        pl.BlockSpec((1, BN, D), _part1),
        pl.BlockSpec((1, BN, 1), _part0),
        pl.BlockSpec((1, BN, 1), _part1),
        pl.BlockSpec((BN, D), lambda i: (i, 0)),
    ],
    out_specs=pl.BlockSpec((BN, D), lambda i: (i, 0)),
    out_shape=jax.ShapeDtypeStruct((N, D), jnp.float32),
)


def kernel(x, edge_index, W1l, b1, W1r, W2l, b2, W2r):
    src = edge_index[0]
    dst = edge_index[1]
    pad = E_PAD - E
    srcp = jnp.concatenate([src, jnp.zeros((pad,), jnp.int32)])
    dstp = jnp.concatenate([dst, jnp.full((pad,), N, jnp.int32)])
    srcm = srcp.reshape(E_PAD // CR, CR)
    dstm = dstp.reshape(E_PAD // CR, CR)
    dstm128 = dstp.reshape(E_PAD // 128, 128)

    xl, xr = _mm2(x, W1l.T, W1r.T, b1.reshape(1, D))
    c_all = _sc_kernel("count")(dstm128)
    s_all = _sc_kernel("edge")(xl, srcm, dstm)
    cn3 = c_all.reshape(NC, N_PAD, 1)
    hl, hr = _fuse(s_all, s_all, cn3, cn3, xr, W2l.T, W2r.T,
                   b2.reshape(1, D))
    t_all = _sc_kernel("edge")(hl, srcm, dstm)
    return _outk(t_all, t_all, cn3, cn3, hr)


# speedup vs baseline: 1.1778x; 1.0001x over previous
import jax
import jax.numpy as jnp
from jax import lax
from jax.experimental import pallas as pl
from jax.experimental.pallas import tpu as pltpu
from jax.experimental.pallas import tpu_sc as plsc

N = 10000
D = 128
E = 320000

NC = 2
NS = 16
NW = NC * NS
CH = 80
CR = 64
GG = 16
NBUF = 4
CW0 = 192
CW1 = 128
EPW = CH * 128
E_PAD = NW * EPW
N_PAD = 10240
RPT = N_PAD // NS
G = 8
BN = 2000


def _zero_rows_f32(ref, ncols):
    def zr(r, _):
        def zc(k, _):
            ref[r, pl.ds(k * 16, 16)] = jnp.zeros((16,), jnp.float32)
            return 0
        return lax.fori_loop(0, ncols // 16, zc, 0)
    lax.fori_loop(0, ref.shape[0], zr, 0)


def _zero_shared_slice(zbuf, shared, base):
    nfull = RPT // 128
    for t in range(nfull):
        pltpu.sync_copy(zbuf, shared.at[pl.ds(base + t * 128, 128)])
    rem = RPT - nfull * 128
    if rem:
        pltpu.sync_copy(zbuf.at[pl.ds(0, rem)],
                        shared.at[pl.ds(base + nfull * 128, rem)])


def _make_edge_kernel():
    mesh = plsc.VectorSubcoreMesh(core_axis_name="c", subcore_axis_name="s",
                                  num_cores=NC, num_subcores=NS)
    out_type = jax.ShapeDtypeStruct((NC, N_PAD, D), jnp.float32)
    scratch = [
        pltpu.VMEM((GG, CR), jnp.int32),
        pltpu.VMEM((GG, CR), jnp.int32),
        [pltpu.VMEM((CR, D), jnp.float32)] * NBUF,
        [pltpu.SemaphoreType.DMA] * NBUF,
        pltpu.VMEM_SHARED((N_PAD, D), jnp.float32),
    ]

    def body(xfeat, srcm, dstm, out, sidx, didx, bufs, sems, acc):
        c = lax.axis_index("c")
        s = lax.axis_index("s")
        base = s * RPT

        _zero_rows_f32(bufs[0], D)
        for t in range(RPT // CR):
            pltpu.sync_copy(bufs[0], acc.at[pl.ds(base + t * CR, CR)])
        plsc.subcore_barrier()

        nch = lax.select(c == 0, CW0 // GG, CW1 // GG)
        cbase = lax.select(c == 0, s * CW0, NS * CW0 + s * CW1)

        def gbody(g, _):
            pltpu.sync_copy(srcm.at[pl.ds(cbase + g * GG, GG)], sidx)
            pltpu.sync_copy(dstm.at[pl.ds(cbase + g * GG, GG)], didx)
            for p in range(NBUF - 1):
                pltpu.async_copy(xfeat.at[sidx.at[p]], bufs[p], sems[p])
            for k in range(GG):
                b = k % NBUF
                pltpu.make_async_copy(xfeat.at[sidx.at[k]], bufs[b],
                                      sems[b]).wait()
                if k + NBUF - 1 < GG:
                    nb = (k + NBUF - 1) % NBUF
                    pltpu.async_copy(xfeat.at[sidx.at[k + NBUF - 1]],
                                     bufs[nb], sems[nb])
                pltpu.sync_copy(bufs[b], acc.at[didx.at[k]], add=True)
            return 0
        lax.fori_loop(0, nch, gbody, 0)
        plsc.subcore_barrier()

        sl = pl.ds(base, RPT)
        pltpu.sync_copy(acc.at[sl], out.at[c, sl])

    cp = pltpu.CompilerParams(needs_layout_passes=False)
    return pl.kernel(body, out_type=out_type, mesh=mesh,
                     scratch_types=scratch, compiler_params=cp)


def _make_count_kernel():
    mesh = plsc.VectorSubcoreMesh(core_axis_name="c", subcore_axis_name="s",
                                  num_cores=NC, num_subcores=NS)
    out_type = jax.ShapeDtypeStruct((NC, 1, N_PAD), jnp.float32)
    scratch = [
        pltpu.VMEM((G, 128), jnp.int32),
        pltpu.VMEM((N_PAD,), jnp.float32),
        pltpu.VMEM((NS, 128), jnp.float32),
        pltpu.VMEM_SHARED((NS, N_PAD), jnp.float32),
    ]

    def body(dstm, cnt_out, didx, cntv, tbuf, sh):
        c = lax.axis_index("c")
        s = lax.axis_index("s")
        w = c * NS + s

        def zc1(k, _):
            cntv[pl.ds(k * 16, 16)] = jnp.zeros((16,), jnp.float32)
            return 0
        lax.fori_loop(0, N_PAD // 16, zc1, 0)

        one16 = jnp.ones((16,), jnp.float32)

        def gbody(g, _):
            pltpu.sync_copy(dstm.at[pl.ds(w * CH + g * G, G)], didx)

            def ebody(j, _):
                for k in range(128 // 16):
                    idx16 = didx[j, pl.ds(k * 16, 16)]
                    plsc.addupdate_scatter(cntv, [idx16], one16)
                return 0
            lax.fori_loop(0, G, ebody, 0)
            return 0
        lax.fori_loop(0, CH // G, gbody, 0)

        pltpu.sync_copy(cntv, sh.at[s])
        plsc.subcore_barrier()
        for h in range(5):
            pltpu.sync_copy(sh.at[:, pl.ds(s * 640 + h * 128, 128)], tbuf)
            for g in range(8):
                csl = pl.ds(g * 16, 16)
                v = tbuf[0, csl]
                for r in range(1, NS):
                    v = v + tbuf[r, csl]
                cntv[pl.ds(h * 128 + g * 16, 16)] = v
        pltpu.sync_copy(cntv.at[pl.ds(0, 640)],
                        cnt_out.at[c, 0, pl.ds(s * 640, 640)])

    cp = pltpu.CompilerParams(needs_layout_passes=False)
    return pl.kernel(body, out_type=out_type, mesh=mesh,
                     scratch_types=scratch, compiler_params=cp)


_sc_kernel_cache = {}


def _sc_kernel(kind):
    if kind not in _sc_kernel_cache:
        maker = {"edge": _make_edge_kernel, "count": _make_count_kernel}
        _sc_kernel_cache[kind] = maker[kind]()
    return _sc_kernel_cache[kind]


def _mm2_body(x_ref, wl_ref, wr_ref, b_ref, yl_ref, yr_ref):
    xb = x_ref[...]
    yl_ref[...] = jnp.dot(xb, wl_ref[...], preferred_element_type=jnp.float32)
    yr_ref[...] = (jnp.dot(xb, wr_ref[...], preferred_element_type=jnp.float32)
                   + b_ref[...])


_mm2 = pl.pallas_call(
    _mm2_body,
    grid=(N // BN,),
    in_specs=[
        pl.BlockSpec((BN, D), lambda i: (i, 0)),
        pl.BlockSpec((D, D), lambda i: (0, 0)),
        pl.BlockSpec((D, D), lambda i: (0, 0)),
        pl.BlockSpec((1, D), lambda i: (0, 0)),
    ],
    out_specs=[pl.BlockSpec((BN, D), lambda i: (i, 0))] * 2,
    out_shape=[jax.ShapeDtypeStruct((N, D), jnp.float32)] * 2,
)


def _part0(i):
    return (0, i, 0)


def _part1(i):
    return (1, i, 0)


def _fuse_body(s0, s1, cn0, cn1, xr, wl, wr, b, hl_ref, hr_ref):
    cnt = cn0[0] + cn1[0]
    recip = 1.0 / jnp.maximum(cnt, 1.0)
    h = jnp.maximum((s0[0] + s1[0]) * recip + xr[...], 0.0)
    hl_ref[...] = jnp.dot(h, wl[...], preferred_element_type=jnp.float32)
    hr_ref[...] = (jnp.dot(h, wr[...], preferred_element_type=jnp.float32)
                   + b[...])


_fuse = pl.pallas_call(
    _fuse_body,
    grid=(N // BN,),
    in_specs=[
        pl.BlockSpec((1, BN, D), _part0),
        pl.BlockSpec((1, BN, D), _part1),
        pl.BlockSpec((1, BN, 1), _part0),
        pl.BlockSpec((1, BN, 1), _part1),
        pl.BlockSpec((BN, D), lambda i: (i, 0)),
        pl.BlockSpec((D, D), lambda i: (0, 0)),
        pl.BlockSpec((D, D), lambda i: (0, 0)),
        pl.BlockSpec((1, D), lambda i: (0, 0)),
    ],
    out_specs=[pl.BlockSpec((BN, D), lambda i: (i, 0))] * 2,
    out_shape=[jax.ShapeDtypeStruct((N, D), jnp.float32)] * 2,
)


def _out_body(s0, s1, cn0, cn1, hr, o_ref):
    cnt = cn0[0] + cn1[0]
    recip = 1.0 / jnp.maximum(cnt, 1.0)
    v = (s0[0] + s1[0]) * recip + hr[...]
    z = v - jnp.max(v, axis=1, keepdims=True)
    o_ref[...] = z - jnp.log(jnp.sum(jnp.exp(z), axis=1, keepdims=True))


_outk = pl.pallas_call(
    _out_body,
    grid=(N // BN,),
    in_specs=[
        pl.BlockSpec((1, BN, D), _part0),
        pl.BlockSpec((1, BN, D), _part1),
        pl.BlockSpec((1, BN, 1), _part0),
        pl.BlockSpec((1, BN, 1), _part1),
        pl.BlockSpec((BN, D), lambda i: (i, 0)),
    ],
    out_specs=pl.BlockSpec((BN, D), lambda i: (i, 0)),
    out_shape=jax.ShapeDtypeStruct((N, D), jnp.float32),
)


def kernel(x, edge_index, W1l, b1, W1r, W2l, b2, W2r):
    src = edge_index[0]
    dst = edge_index[1]
    pad = E_PAD - E
    srcp = jnp.concatenate([src, jnp.zeros((pad,), jnp.int32)])
    dstp = jnp.concatenate([dst, jnp.full((pad,), N, jnp.int32)])
    srcm = srcp.reshape(E_PAD // CR, CR)
    dstm = dstp.reshape(E_PAD // CR, CR)
    dstm128 = dstp.reshape(E_PAD // 128, 128)

    xl, xr = _mm2(x, W1l.T, W1r.T, b1.reshape(1, D))
    c_all = _sc_kernel("count")(dstm128)
    s_all = _sc_kernel("edge")(xl, srcm, dstm)
    cn3 = c_all.reshape(NC, N_PAD, 1)
    hl, hr = _fuse(s_all, s_all, cn3, cn3, xr, W2l.T, W2r.T,
                   b2.reshape(1, D))
    t_all = _sc_kernel("edge")(hl, srcm, dstm)
    return _outk(t_all, t_all, cn3, cn3, hr)
